# Initial kernel scaffold; baseline (speedup 1.0000x reference)
import jax
import jax.numpy as jnp
from jax.experimental import pallas as pl


def kernel(x, edges, layers_lengths, W1, b1, W2, b2, W3, b3, Wc, bc):
    raise NotImplementedError("write your pallas kernel here")


# speedup vs baseline: 10.8509x; 10.8509x over previous
import functools

import jax
import jax.numpy as jnp
from jax import lax
from jax.experimental import pallas as pl
from jax.experimental.pallas import tpu as pltpu
from jax.experimental.pallas import tpu_sc as plsc

N_ = 10000
E_ = 160000
NTILES = 16
EPT = E_ // NTILES
CH = 80
NCHUNK = EPT // CH
ROWS_A = 640
ROWS_LAST = N_ - 15 * ROWS_A

_MESH = plsc.VectorSubcoreMesh(core_axis_name="c", subcore_axis_name="s")


def _zero_buf(buf, rows, groups):
    def body(i, _):
        r = i // groups
        g = i - r * groups
        buf[r, pl.ds(g * 16, 16)] = jnp.zeros((16,), jnp.float32)
        return 0
    lax.fori_loop(0, rows * groups, body, 0)


def _make_prop(F):
    G = F // 16

    @functools.partial(
        pl.kernel,
        out_type=jax.ShapeDtypeStruct((2 * N_, F), jnp.float32),
        mesh=_MESH,
        scratch_types=[
            pltpu.VMEM((EPT,), jnp.int32),
            pltpu.VMEM((NCHUNK, CH), jnp.int32),
            pltpu.VMEM((CH, F), jnp.float32),
            pltpu.VMEM((CH, F), jnp.float32),
            pltpu.VMEM_SHARED((N_, F), jnp.float32),
            pltpu.SemaphoreType.DMA,
        ],
    )
    def prop(y_hbm, srcF, dstR, z_hbm, src_v, dst_v, buf, zb, acc, sem):
        c = lax.axis_index("c")
        s = lax.axis_index("s")

        pltpu.sync_copy(srcF.at[s], src_v)
        pltpu.sync_copy(dstR.at[s], dst_v)

        coff = c * N_
        def add_off(i, _):
            src_v[pl.ds(i * 16, 16)] = src_v[pl.ds(i * 16, 16)] + coff
            return 0
        lax.fori_loop(0, EPT // 16, add_off, 0)

        _zero_buf(zb, CH, G)

        @pl.when(s < 15)
        def _():
            def zcp(i, _):
                pltpu.sync_copy(zb, acc.at[pl.ds(s * ROWS_A + i * CH, CH)])
                return 0
            lax.fori_loop(0, ROWS_A // CH, zcp, 0)

        @pl.when(s == 15)
        def _():
            def zcp(i, _):
                pltpu.sync_copy(zb, acc.at[pl.ds(15 * ROWS_A + i * CH, CH)])
                return 0
            lax.fori_loop(0, ROWS_LAST // CH, zcp, 0)

        plsc.subcore_barrier()

        def chunk(k, _):
            pltpu.async_copy(y_hbm.at[src_v.at[pl.ds(k * CH, CH)]], buf, sem).wait()
            pltpu.sync_copy(buf, acc.at[dst_v.at[k]], add=True)
            return 0
        lax.fori_loop(0, NCHUNK, chunk, 0)

        plsc.subcore_barrier()

        @pl.when(s < 15)
        def _():
            pltpu.sync_copy(acc.at[pl.ds(s * ROWS_A, ROWS_A)],
                            z_hbm.at[pl.ds(c * N_ + s * ROWS_A, ROWS_A)])

        @pl.when(s == 15)
        def _():
            pltpu.sync_copy(acc.at[pl.ds(15 * ROWS_A, ROWS_LAST)],
                            z_hbm.at[pl.ds(c * N_ + 15 * ROWS_A, ROWS_LAST)])

    return prop


_prop128 = _make_prop(128)


@functools.partial(
    pl.kernel,
    out_type=jax.ShapeDtypeStruct((2 * N_, 16), jnp.float32),
    mesh=_MESH,
    scratch_types=[
        pltpu.VMEM((NCHUNK, CH), jnp.int32),
        pltpu.VMEM((CH, 16), jnp.float32),
        pltpu.VMEM((CH, 16), jnp.float32),
        pltpu.VMEM_SHARED((N_, 16), jnp.float32),
        pltpu.SemaphoreType.DMA,
    ],
)
def _deg_kernel(dstR, out_hbm, dst_v, ones, zb, acc, sem):
    c = lax.axis_index("c")
    s = lax.axis_index("s")

    pltpu.sync_copy(dstR.at[s], dst_v)

    def fill(i, _):
        ones[i, :] = jnp.ones((16,), jnp.float32)
        zb[i, :] = jnp.zeros((16,), jnp.float32)
        return 0
    lax.fori_loop(0, CH, fill, 0)

    @pl.when(s < 15)
    def _():
        def zcp(i, _):
            pltpu.sync_copy(zb, acc.at[pl.ds(s * ROWS_A + i * CH, CH)])
            return 0
        lax.fori_loop(0, ROWS_A // CH, zcp, 0)

    @pl.when(s == 15)
    def _():
        def zcp(i, _):
            pltpu.sync_copy(zb, acc.at[pl.ds(15 * ROWS_A + i * CH, CH)])
            return 0
        lax.fori_loop(0, ROWS_LAST // CH, zcp, 0)

    plsc.subcore_barrier()

    def chunk(k, _):
        pltpu.sync_copy(ones, acc.at[dst_v.at[k]], add=True)
        return 0
    lax.fori_loop(0, NCHUNK, chunk, 0)

    plsc.subcore_barrier()

    @pl.when(s < 15)
    def _():
        pltpu.sync_copy(acc.at[pl.ds(s * ROWS_A, ROWS_A)],
                        out_hbm.at[pl.ds(c * N_ + s * ROWS_A, ROWS_A)])

    @pl.when(s == 15)
    def _():
        pltpu.sync_copy(acc.at[pl.ds(15 * ROWS_A, ROWS_LAST)],
                        out_hbm.at[pl.ds(c * N_ + 15 * ROWS_A, ROWS_LAST)])


NB = 400
GRID = N_ // NB


def _dinv_of(deg_ref):
    return lax.rsqrt(deg_ref[:, 0:1] + 1.0)


def _tc1_body(deg_ref, x_ref, y_ref):
    dinv = _dinv_of(deg_ref)
    y = x_ref[...] * dinv
    y_ref[0] = y[:, :128]
    y_ref[1] = y[:, 128:]


def _tc2_body(deg_ref, z_ref, y_ref, w1_ref, b1_ref, w2_ref, y2_ref):
    dinv = _dinv_of(deg_ref)
    a = dinv * (jnp.concatenate([z_ref[0], z_ref[1]], axis=1)
                + jnp.concatenate([y_ref[0], y_ref[1]], axis=1))
    h1 = jnp.dot(a, w1_ref[...], preferred_element_type=jnp.float32) + b1_ref[...]
    h1 = jnp.clip(h1, 0.0, 6.0)
    u2 = jnp.dot(h1, w2_ref[...], preferred_element_type=jnp.float32)
    y2 = dinv * u2
    y2_ref[0] = y2[:, :128]
    y2_ref[1] = y2[:, 128:]


def _tc3_body(deg_ref, z_ref, y_ref, b2_ref, w3_ref, y3_ref):
    dinv = _dinv_of(deg_ref)
    h2 = dinv * (jnp.concatenate([z_ref[0], z_ref[1]], axis=1)
                 + jnp.concatenate([y_ref[0], y_ref[1]], axis=1)) + b2_ref[...]
    h2 = jnp.clip(h2, 0.0, 6.0)
    u3 = jnp.dot(h2, w3_ref[...], preferred_element_type=jnp.float32)
    y3 = dinv * u3
    y3_ref[0] = y3[:, :128]
    y3_ref[1] = y3[:, 128:]


def _tc4_body(deg_ref, z_ref, y_ref, b3_ref, wc_ref, bc_ref, o_ref):
    dinv = _dinv_of(deg_ref)
    v = dinv * (jnp.concatenate([z_ref[0], z_ref[1]], axis=1)
                + jnp.concatenate([y_ref[0], y_ref[1]], axis=1)) + b3_ref[...]
    h3 = jnp.clip(v, 0.0, 6.0)
    o = jnp.dot(h3, wc_ref[...], preferred_element_type=jnp.float32) + bc_ref[...]
    o_ref[...] = jax.nn.sigmoid(o)


def _nodes(shape2):
    return pl.BlockSpec((NB, shape2), lambda i: (i, 0))


def _halves(F):
    return pl.BlockSpec((2, NB, F), lambda i: (0, i, 0))


def _full(a, b):
    return pl.BlockSpec((a, b), lambda i: (0, 0))


def kernel(x, edges, layers_lengths, W1, b1, W2, b2, W3, b3, Wc, bc):
    srcF = edges[0].reshape(NTILES, EPT)
    dstR = edges[1].reshape(NTILES, NCHUNK, CH)

    deg16 = _deg_kernel(dstR)[:N_]

    y1 = pl.pallas_call(
        _tc1_body,
        grid=(GRID,),
        in_specs=[_nodes(16), _nodes(256)],
        out_specs=_halves(128),
        out_shape=jax.ShapeDtypeStruct((2, N_, 128), jnp.float32),
    )(deg16, x)

    z1 = _prop128(y1.reshape(2 * N_, 128), srcF, dstR).reshape(2, N_, 128)

    y2 = pl.pallas_call(
        _tc2_body,
        grid=(GRID,),
        in_specs=[_nodes(16), _halves(128), _halves(128),
                  _full(256, 512), _full(1, 512), _full(512, 256)],
        out_specs=_halves(128),
        out_shape=jax.ShapeDtypeStruct((2, N_, 128), jnp.float32),
    )(deg16, z1, y1, W1, b1.reshape(1, 512), W2)

    z2 = _prop128(y2.reshape(2 * N_, 128), srcF, dstR).reshape(2, N_, 128)

    W3p = jnp.pad(W3, ((0, 0), (0, 204)))
    y3 = pl.pallas_call(
        _tc3_body,
        grid=(GRID,),
        in_specs=[_nodes(16), _halves(128), _halves(128),
                  _full(1, 256), _full(256, 256)],
        out_specs=_halves(128),
        out_shape=jax.ShapeDtypeStruct((2, N_, 128), jnp.float32),
    )(deg16, z2, y2, b2.reshape(1, 256), W3p)

    z3 = _prop128(y3.reshape(2 * N_, 128), srcF, dstR).reshape(2, N_, 128)

    b3p = jnp.pad(b3, (0, 204)).reshape(1, 256)
    Wcp = jnp.pad(Wc, ((0, 204), (0, 0)))
    out = pl.pallas_call(
        _tc4_body,
        grid=(GRID,),
        in_specs=[_nodes(16), _halves(128), _halves(128),
                  _full(1, 256), _full(256, 3), _full(1, 3)],
        out_specs=_nodes(3),
        out_shape=jax.ShapeDtypeStruct((N_, 3), jnp.float32),
    )(deg16, z3, y3, b3p, Wcp, bc.reshape(1, 3))

    return (out, edges, layers_lengths)
